# baseline (speedup 1.0000x reference)
import functools

import jax
import jax.numpy as jnp
from jax import lax
from jax.experimental import pallas as pl
from jax.experimental.pallas import tpu as pltpu
from jax.experimental.pallas import tpu_sc as plsc

DIM = 32
SLICE = 128
NW = 32
NC = 2


def _rsqrt_vec(v):
    i = lax.bitcast_convert_type(v, jnp.int32)
    i = jnp.int32(0x5F3759DF) - lax.shift_right_logical(i, 1)
    y = lax.bitcast_convert_type(i, jnp.float32)
    y = y * (1.5 - 0.5 * v * y * y)
    y = y * (1.5 - 0.5 * v * y * y)
    y = y * (1.5 - 0.5 * v * y * y)
    return y


_GATHER_DNUMS = lax.GatherDimensionNumbers(
    offset_dims=(), collapsed_slice_dims=(0,), start_index_map=(0,))


def _lane_perm(v, p):
    return lax.gather(v, p[:, None], _GATHER_DNUMS, (1,),
                      mode=lax.GatherScatterMode.PROMISE_IN_BOUNDS)


def _lane_sum(v, perms):
    for p in perms:
        v = v + _lane_perm(v, p)
    return v


def _make_sc_kernel(n_rows):
    assert n_rows % (NW * SLICE) == 0
    slices_per_w = n_rows // (NW * SLICE)
    mesh = plsc.VectorSubcoreMesh(core_axis_name="c", subcore_axis_name="s")

    @functools.partial(
        pl.kernel,
        mesh=mesh,
        out_type=jax.ShapeDtypeStruct((n_rows, DIM), jnp.float32),
        scratch_types=[
            pltpu.VMEM((SLICE,), jnp.int32),
            pltpu.VMEM((SLICE, DIM), jnp.float32),
            pltpu.VMEM((DIM,), jnp.float32),
            pltpu.VMEM((DIM,), jnp.float32),
            pltpu.SemaphoreType.DMA,
        ],
        compiler_params=pltpu.CompilerParams(use_tc_tiling_on_sc=False),
    )
    def body(x2d, table, gamma, beta, out, idx_v, rows_v, g_v, b_v, sem):
        c = lax.axis_index("c")
        s = lax.axis_index("s")
        wid = s * NC + c
        pltpu.sync_copy(gamma, g_v)
        pltpu.sync_copy(beta, b_v)
        g0 = g_v[pl.ds(0, 16)]
        g1 = g_v[pl.ds(16, 16)]
        b0 = b_v[pl.ds(0, 16)]
        b1 = b_v[pl.ds(16, 16)]
        lanes = lax.iota(jnp.int32, 16)
        perms = [jnp.bitwise_xor(lanes, jnp.int32(k)) for k in (1, 2, 4, 8)]
        base = wid * slices_per_w

        def chunk_body(j, carry):
            row0 = base + j
            pltpu.sync_copy(x2d.at[row0], idx_v)
            pltpu.async_copy(table.at[idx_v], rows_v, sem).wait()

            def row_body(r, carry2):
                v0 = rows_v[r, pl.ds(0, 16)]
                v1 = rows_v[r, pl.ds(16, 16)]
                sm = _lane_sum(v0 + v1, perms)
                sq = _lane_sum(v0 * v0 + v1 * v1, perms)
                mu = sm * (1.0 / DIM)
                var = sq * (1.0 / DIM) - mu * mu
                var = jnp.maximum(var, 0.0) + 1e-5
                rs = _rsqrt_vec(var)
                rows_v[r, pl.ds(0, 16)] = (v0 - mu) * (rs * g0) + b0
                rows_v[r, pl.ds(16, 16)] = (v1 - mu) * (rs * g1) + b1
                return carry2

            lax.fori_loop(0, SLICE, row_body, 0, unroll=4)
            pltpu.sync_copy(rows_v, out.at[pl.ds(row0 * SLICE, SLICE)])
            return carry

        lax.fori_loop(0, slices_per_w, chunk_body, 0)

    return body


def kernel(x, table, gamma, beta):
    b, h = x.shape
    n_rows = b * h
    x2d = x.reshape(-1).astype(jnp.int32).reshape(n_rows // SLICE, SLICE)
    out = _make_sc_kernel(n_rows)(x2d, table, gamma, beta)
    return out.reshape(b, h, DIM)
